# baseline (speedup 1.0000x reference)
import functools

import jax
import jax.numpy as jnp
from jax import lax
from jax.experimental import pallas as pl
from jax.experimental.pallas import tpu as pltpu
from jax.experimental.pallas import tpu_sc as plsc

H = 1024
E = 8
FF = 1024
SFF = 2048
T = 2048
TB = 512
MB = 256
NBLK = (T * 2) // MB + E
PAD_T = NBLK * MB
EPS = 1e-6
SCALING = 1.0

F32 = jnp.float32
BF16 = jnp.bfloat16

NC = 2
NS = 16
NW = NC * NS
CH = T // NW
CHG = CH // 2


def _prologue_body(x_ref, nw_ref, gw_ref, h_ref, meta_ref, cnt_ref, carry):
    b = pl.program_id(0)

    @pl.when(b == 0)
    def _():
        carry[...] = jnp.zeros_like(carry)

    x = x_ref[...]
    var = jnp.mean(x * x, axis=1, keepdims=True)
    h = x * lax.rsqrt(var + EPS) * nw_ref[...]
    h_ref[...] = h

    logits = lax.dot_general(h, gw_ref[...], (((1,), (1,)), ((), ())),
                             precision=lax.Precision.HIGHEST)
    lane = lax.broadcasted_iota(jnp.int32, (TB, E), 1)
    m1 = jnp.max(logits, axis=1, keepdims=True)
    i1 = jnp.min(jnp.where(logits == m1, lane, E), axis=1, keepdims=True)
    oh1 = lane == i1
    l2 = jnp.where(oh1, -jnp.inf, logits)
    m2 = jnp.max(l2, axis=1, keepdims=True)
    i2 = jnp.min(jnp.where(l2 == m2, lane, E), axis=1, keepdims=True)
    oh2 = lane == i2

    t = jnp.exp(m2 - m1)
    w1 = (1.0 / (1.0 + t)) * SCALING
    w2 = (t / (1.0 + t)) * SCALING

    ohc = oh1.astype(F32) + oh2.astype(F32)
    r = lax.broadcasted_iota(jnp.int32, (TB, TB), 0)
    c = lax.broadcasted_iota(jnp.int32, (TB, TB), 1)
    strict = (r > c).astype(F32)
    sex = lax.dot_general(strict, ohc, (((1,), (0,)), ((), ()))) + carry[...]
    prior1 = jnp.sum(jnp.where(oh1, sex, 0.0), axis=1, keepdims=True)
    prior2 = jnp.sum(jnp.where(oh2, sex, 0.0), axis=1, keepdims=True)
    carry[...] = carry[...] + jnp.sum(ohc, axis=0, keepdims=True)
    cnt_ref[...] = carry[...]

    meta = jnp.zeros((TB, E), F32)
    for j, v in enumerate((i1.astype(F32), i2.astype(F32),
                           prior1, prior2, w1, w2)):
        meta = jnp.where(lane == j, v, meta)
    meta_ref[...] = meta


def _prologue(x, nw, gw):
    return pl.pallas_call(
        _prologue_body,
        grid=(T // TB,),
        in_specs=[
            pl.BlockSpec((TB, H), lambda b: (b, 0)),
            pl.BlockSpec((1, H), lambda b: (0, 0)),
            pl.BlockSpec((E, H), lambda b: (0, 0)),
        ],
        out_specs=[
            pl.BlockSpec((TB, H), lambda b: (b, 0)),
            pl.BlockSpec((TB, E), lambda b: (b, 0)),
            pl.BlockSpec((1, E), lambda b: (0, 0)),
        ],
        out_shape=[
            jax.ShapeDtypeStruct((T, H), F32),
            jax.ShapeDtypeStruct((T, E), F32),
            jax.ShapeDtypeStruct((1, E), F32),
        ],
        scratch_shapes=[pltpu.VMEM((1, E), F32)],
    )(x, nw, gw)


def _route_body(meta_ref, cnt_ref, dest_ref, be_ref):
    lane8 = lax.broadcasted_iota(jnp.int32, (T, E), 1)
    bstart = lax.broadcasted_iota(jnp.int32, (1, NBLK), 1) * MB
    padded_row = jnp.zeros((T, E), F32)
    nblocks = jnp.zeros((1, NBLK), jnp.int32)
    run = 0
    for g in range(E):
        cg = cnt_ref[0, g].astype(jnp.int32)
        pg = ((cg + (MB - 1)) // MB) * MB
        padded_row = jnp.where(lane8 == g, pg.astype(F32), padded_row)
        nblocks = nblocks + jnp.where(run <= bstart, 1, 0)
        run = run + pg
    be_ref[...] = jnp.maximum(nblocks - 1, 0)

    meta = meta_ref[...]
    lane = lane8.astype(F32)
    i1 = meta[:, 0:1]
    i2 = meta[:, 1:2]
    b1 = jnp.sum(jnp.where(lane < i1, padded_row, 0.0), axis=1, keepdims=True)
    b2 = jnp.sum(jnp.where(lane < i2, padded_row, 0.0), axis=1, keepdims=True)
    d1 = b1 + meta[:, 2:3]
    d2 = b2 + meta[:, 3:4]
    lane2 = lax.broadcasted_iota(jnp.int32, (T, 2), 1)
    dest_ref[...] = jnp.where(lane2 == 0, d1, d2).astype(jnp.int32)


def _route(meta, cnt):
    return pl.pallas_call(
        _route_body,
        grid=(1,),
        in_specs=[
            pl.BlockSpec((T, E), lambda i: (0, 0)),
            pl.BlockSpec(memory_space=pltpu.SMEM),
        ],
        out_specs=[
            pl.BlockSpec((T, 2), lambda i: (0, 0)),
            pl.BlockSpec((1, NBLK), lambda i: (0, 0)),
        ],
        out_shape=[
            jax.ShapeDtypeStruct((T, 2), jnp.int32),
            jax.ShapeDtypeStruct((1, NBLK), jnp.int32),
        ],
    )(meta, cnt)


def _sc_scatter(h, d1, d2):
    mesh = plsc.VectorSubcoreMesh(core_axis_name="c", subcore_axis_name="s")

    @functools.partial(
        pl.kernel,
        mesh=mesh,
        out_type=jax.ShapeDtypeStruct((PAD_T, H), F32),
        scratch_types=[
            pltpu.VMEM((CH,), jnp.int32),
            pltpu.VMEM((CH,), jnp.int32),
            pltpu.VMEM((CH, H), F32),
            pltpu.SemaphoreType.DMA,
            pltpu.SemaphoreType.DMA,
        ],
    )
    def k(h_hbm, d1_hbm, d2_hbm, hs_hbm, i1_v, i2_v, rows_v, sem, sem2):
        wid = lax.axis_index("s") * NC + lax.axis_index("c")
        base = wid * CH
        pltpu.sync_copy(d1_hbm.at[pl.ds(base, CH)], i1_v)
        pltpu.sync_copy(d2_hbm.at[pl.ds(base, CH)], i2_v)
        pltpu.sync_copy(h_hbm.at[pl.ds(base, CH)], rows_v)
        cp1 = pltpu.async_copy(rows_v, hs_hbm.at[i1_v], sem)
        cp2 = pltpu.async_copy(rows_v, hs_hbm.at[i2_v], sem2)
        cp1.wait()
        cp2.wait()

    return k(h, d1, d2)


def _sc_gather(y, d1, d2):
    mesh = plsc.VectorSubcoreMesh(core_axis_name="c", subcore_axis_name="s")

    @functools.partial(
        pl.kernel,
        mesh=mesh,
        out_type=[
            jax.ShapeDtypeStruct((T, H), F32),
            jax.ShapeDtypeStruct((T, H), F32),
        ],
        scratch_types=[
            pltpu.VMEM((CHG,), jnp.int32),
            pltpu.VMEM((CHG,), jnp.int32),
            pltpu.VMEM((CHG, H), F32),
            pltpu.VMEM((CHG, H), F32),
            pltpu.SemaphoreType.DMA,
            pltpu.SemaphoreType.DMA,
        ],
    )
    def k(y_hbm, d1_hbm, d2_hbm, o1_hbm, o2_hbm, i1_v, i2_v, r1_v, r2_v,
          sem, sem2):
        wid = lax.axis_index("s") * NC + lax.axis_index("c")

        @pl.loop(0, CH, step=CHG)
        def _(c):
            base = wid * CH + c
            pltpu.sync_copy(d1_hbm.at[pl.ds(base, CHG)], i1_v)
            pltpu.sync_copy(d2_hbm.at[pl.ds(base, CHG)], i2_v)
            cp1 = pltpu.async_copy(y_hbm.at[i1_v], r1_v, sem)
            cp2 = pltpu.async_copy(y_hbm.at[i2_v], r2_v, sem2)
            cp1.wait()
            pltpu.sync_copy(r1_v, o1_hbm.at[pl.ds(base, CHG)])
            cp2.wait()
            pltpu.sync_copy(r2_v, o2_hbm.at[pl.ds(base, CHG)])

    return k(y, d1, d2)


def _shared_body(x_ref, h_ref, su_ref, sd_ref, base_ref):
    h = h_ref[...]
    u = lax.dot_general(h, su_ref[...], (((1,), (1,)), ((), ())),
                        preferred_element_type=F32)
    act = jax.nn.silu(u[:, :SFF]) * u[:, SFF:]
    out = lax.dot_general(act, sd_ref[...], (((1,), (1,)), ((), ())),
                          preferred_element_type=F32)
    base_ref[...] = x_ref[...] + out


def _shared(x, h, su, sd, half):
    off = half * (T // TB // 2)
    return pl.pallas_call(
        _shared_body,
        grid=(T // TB // 2,),
        in_specs=[
            pl.BlockSpec((TB, H), lambda b: (b + off, 0)),
            pl.BlockSpec((TB, H), lambda b: (b + off, 0)),
            pl.BlockSpec((2 * SFF, H), lambda b: (0, 0)),
            pl.BlockSpec((H, SFF), lambda b: (0, 0)),
        ],
        out_specs=pl.BlockSpec((TB, H), lambda b: (b, 0)),
        out_shape=jax.ShapeDtypeStruct((T // 2, H), F32),
    )(x, h, su, sd)


def _moe_body(be_ref, hs_ref, up_ref, dn_ref, y_ref):
    hs = hs_ref[...]
    u = lax.dot_general(hs, up_ref[0], (((1,), (1,)), ((), ())),
                        preferred_element_type=F32)
    act = jax.nn.silu(u[:, :FF]) * u[:, FF:]
    y_ref[...] = lax.dot_general(act, dn_ref[0], (((1,), (1,)), ((), ())),
                                 preferred_element_type=F32)


def _moe(be_flat, hs, up_w, down_w):
    grid_spec = pltpu.PrefetchScalarGridSpec(
        num_scalar_prefetch=1,
        grid=(NBLK,),
        in_specs=[
            pl.BlockSpec((MB, H), lambda b, be: (b, 0)),
            pl.BlockSpec((1, 2 * FF, H), lambda b, be: (be[b], 0, 0)),
            pl.BlockSpec((1, H, FF), lambda b, be: (be[b], 0, 0)),
        ],
        out_specs=pl.BlockSpec((MB, H), lambda b, be: (b, 0)),
    )
    return pl.pallas_call(
        _moe_body,
        grid_spec=grid_spec,
        out_shape=jax.ShapeDtypeStruct((PAD_T, H), F32),
    )(be_flat, hs, up_w, down_w)


def _final_body(ba_ref, bb_ref, meta_ref, y1_ref, y2_ref, out_ref):
    b = pl.program_id(0)
    w1 = meta_ref[:, 4:5]
    w2 = meta_ref[:, 5:6]
    comb = w1 * y1_ref[...] + w2 * y2_ref[...]

    @pl.when(b < (T // TB // 2))
    def _():
        out_ref[...] = ba_ref[...] + comb

    @pl.when(b >= (T // TB // 2))
    def _():
        out_ref[...] = bb_ref[...] + comb


def _final(base_a, base_b, meta, y1, y2):
    hg = T // TB // 2
    return pl.pallas_call(
        _final_body,
        grid=(T // TB,),
        in_specs=[
            pl.BlockSpec((TB, H), lambda b: (jnp.minimum(b, hg - 1), 0)),
            pl.BlockSpec((TB, H), lambda b: (jnp.maximum(b, hg) - hg, 0)),
            pl.BlockSpec((TB, E), lambda b: (b, 0)),
            pl.BlockSpec((TB, H), lambda b: (b, 0)),
            pl.BlockSpec((TB, H), lambda b: (b, 0)),
        ],
        out_specs=pl.BlockSpec((TB, H), lambda b: (b, 0)),
        out_shape=jax.ShapeDtypeStruct((T, H), F32),
    )(base_a, base_b, meta, y1, y2)


def kernel(x, ffn_norm_w, gate_w, up_w, down_w, shared_up_w, shared_down_w):
    nw = ffn_norm_w.reshape(1, H)
    h, meta, cnt = _prologue(x, nw, gate_w)
    dest, be = _route(meta, cnt)
    d1 = dest[:, 0]
    d2 = dest[:, 1]
    be_flat = be.reshape(NBLK)
    hs = _sc_scatter(h, d1, d2)
    base_a = _shared(x, h, shared_up_w, shared_down_w, 0)
    y = _moe(be_flat, hs, up_w, down_w)
    y1, y2 = _sc_gather(y, d1, d2)
    base_b = _shared(x, h, shared_up_w, shared_down_w, 1)
    return _final(base_a, base_b, meta, y1, y2)


# speedup vs baseline: 1.0241x; 1.0241x over previous
import functools

import jax
import jax.numpy as jnp
from jax import lax
from jax.experimental import pallas as pl
from jax.experimental.pallas import tpu as pltpu
from jax.experimental.pallas import tpu_sc as plsc

H = 1024
E = 8
FF = 1024
SFF = 2048
T = 2048
TB = 512
MB = 256
NBLK = (T * 2) // MB + E
PAD_T = NBLK * MB
EPS = 1e-6
SCALING = 1.0

F32 = jnp.float32
BF16 = jnp.bfloat16

NC = 2
NS = 16
NW = NC * NS
CH = T // NW
CHG = CH // 2


def _prologue_body(x_ref, nw_ref, gw_ref, h_ref, meta_ref, cnt_ref, carry):
    b = pl.program_id(0)

    @pl.when(b == 0)
    def _():
        carry[...] = jnp.zeros_like(carry)

    x = x_ref[...]
    var = jnp.mean(x * x, axis=1, keepdims=True)
    h = x * lax.rsqrt(var + EPS) * nw_ref[...]
    h_ref[...] = h

    logits = lax.dot_general(h, gw_ref[...], (((1,), (1,)), ((), ())),
                             precision=lax.Precision.HIGHEST)
    lane = lax.broadcasted_iota(jnp.int32, (TB, E), 1)
    m1 = jnp.max(logits, axis=1, keepdims=True)
    i1 = jnp.min(jnp.where(logits == m1, lane, E), axis=1, keepdims=True)
    oh1 = lane == i1
    l2 = jnp.where(oh1, -jnp.inf, logits)
    m2 = jnp.max(l2, axis=1, keepdims=True)
    i2 = jnp.min(jnp.where(l2 == m2, lane, E), axis=1, keepdims=True)
    oh2 = lane == i2

    t = jnp.exp(m2 - m1)
    w1 = (1.0 / (1.0 + t)) * SCALING
    w2 = (t / (1.0 + t)) * SCALING

    ohc = oh1.astype(F32) + oh2.astype(F32)
    r = lax.broadcasted_iota(jnp.int32, (TB, TB), 0)
    c = lax.broadcasted_iota(jnp.int32, (TB, TB), 1)
    strict = (r > c).astype(F32)
    sex = lax.dot_general(strict, ohc, (((1,), (0,)), ((), ()))) + carry[...]
    prior1 = jnp.sum(jnp.where(oh1, sex, 0.0), axis=1, keepdims=True)
    prior2 = jnp.sum(jnp.where(oh2, sex, 0.0), axis=1, keepdims=True)
    carry[...] = carry[...] + jnp.sum(ohc, axis=0, keepdims=True)
    cnt_ref[...] = carry[...]

    meta = jnp.zeros((TB, E), F32)
    for j, v in enumerate((i1.astype(F32), i2.astype(F32),
                           prior1, prior2, w1, w2)):
        meta = jnp.where(lane == j, v, meta)
    meta_ref[...] = meta


def _prologue(x, nw, gw):
    return pl.pallas_call(
        _prologue_body,
        grid=(T // TB,),
        in_specs=[
            pl.BlockSpec((TB, H), lambda b: (b, 0)),
            pl.BlockSpec((1, H), lambda b: (0, 0)),
            pl.BlockSpec((E, H), lambda b: (0, 0)),
        ],
        out_specs=[
            pl.BlockSpec((TB, H), lambda b: (b, 0)),
            pl.BlockSpec((TB, E), lambda b: (b, 0)),
            pl.BlockSpec((1, E), lambda b: (0, 0)),
        ],
        out_shape=[
            jax.ShapeDtypeStruct((T, H), F32),
            jax.ShapeDtypeStruct((T, E), F32),
            jax.ShapeDtypeStruct((1, E), F32),
        ],
        scratch_shapes=[pltpu.VMEM((1, E), F32)],
    )(x, nw, gw)


def _route_body(meta_ref, cnt_ref, dest_ref, be_ref):
    lane8 = lax.broadcasted_iota(jnp.int32, (T, E), 1)
    bstart = lax.broadcasted_iota(jnp.int32, (1, NBLK), 1) * MB
    padded_row = jnp.zeros((T, E), F32)
    nblocks = jnp.zeros((1, NBLK), jnp.int32)
    run = 0
    for g in range(E):
        cg = cnt_ref[0, g].astype(jnp.int32)
        pg = ((cg + (MB - 1)) // MB) * MB
        padded_row = jnp.where(lane8 == g, pg.astype(F32), padded_row)
        nblocks = nblocks + jnp.where(run <= bstart, 1, 0)
        run = run + pg
    be_ref[...] = jnp.maximum(nblocks - 1, 0)

    meta = meta_ref[...]
    lane = lane8.astype(F32)
    i1 = meta[:, 0:1]
    i2 = meta[:, 1:2]
    b1 = jnp.sum(jnp.where(lane < i1, padded_row, 0.0), axis=1, keepdims=True)
    b2 = jnp.sum(jnp.where(lane < i2, padded_row, 0.0), axis=1, keepdims=True)
    d1 = b1 + meta[:, 2:3]
    d2 = b2 + meta[:, 3:4]
    lane2 = lax.broadcasted_iota(jnp.int32, (T, 2), 1)
    dest_ref[...] = jnp.where(lane2 == 0, d1, d2).astype(jnp.int32)


def _route(meta, cnt):
    return pl.pallas_call(
        _route_body,
        grid=(1,),
        in_specs=[
            pl.BlockSpec((T, E), lambda i: (0, 0)),
            pl.BlockSpec(memory_space=pltpu.SMEM),
        ],
        out_specs=[
            pl.BlockSpec((T, 2), lambda i: (0, 0)),
            pl.BlockSpec((1, NBLK), lambda i: (0, 0)),
        ],
        out_shape=[
            jax.ShapeDtypeStruct((T, 2), jnp.int32),
            jax.ShapeDtypeStruct((1, NBLK), jnp.int32),
        ],
    )(meta, cnt)


def _sc_scatter(h, d1, d2):
    mesh = plsc.VectorSubcoreMesh(core_axis_name="c", subcore_axis_name="s")

    @functools.partial(
        pl.kernel,
        mesh=mesh,
        out_type=jax.ShapeDtypeStruct((PAD_T, H), F32),
        scratch_types=[
            pltpu.VMEM((CH,), jnp.int32),
            pltpu.VMEM((CH,), jnp.int32),
            pltpu.VMEM((CH, H), F32),
            pltpu.SemaphoreType.DMA,
            pltpu.SemaphoreType.DMA,
        ],
    )
    def k(h_hbm, d1_hbm, d2_hbm, hs_hbm, i1_v, i2_v, rows_v, sem, sem2):
        wid = lax.axis_index("s") * NC + lax.axis_index("c")
        base = wid * CH
        pltpu.sync_copy(d1_hbm.at[pl.ds(base, CH)], i1_v)
        pltpu.sync_copy(d2_hbm.at[pl.ds(base, CH)], i2_v)
        pltpu.sync_copy(h_hbm.at[pl.ds(base, CH)], rows_v)
        cp1 = pltpu.async_copy(rows_v, hs_hbm.at[i1_v], sem)
        cp2 = pltpu.async_copy(rows_v, hs_hbm.at[i2_v], sem2)
        cp1.wait()
        cp2.wait()

    return k(h, d1, d2)


def _sc_gather(y, d1, d2):
    mesh = plsc.VectorSubcoreMesh(core_axis_name="c", subcore_axis_name="s")

    @functools.partial(
        pl.kernel,
        mesh=mesh,
        out_type=[
            jax.ShapeDtypeStruct((T, H), F32),
            jax.ShapeDtypeStruct((T, H), F32),
        ],
        scratch_types=[
            pltpu.VMEM((CHG,), jnp.int32),
            pltpu.VMEM((CHG,), jnp.int32),
            pltpu.VMEM((CHG, H), F32),
            pltpu.VMEM((CHG, H), F32),
            pltpu.SemaphoreType.DMA,
            pltpu.SemaphoreType.DMA,
        ],
    )
    def k(y_hbm, d1_hbm, d2_hbm, o1_hbm, o2_hbm, i1_v, i2_v, r1_v, r2_v,
          sem, sem2):
        wid = lax.axis_index("s") * NC + lax.axis_index("c")

        @pl.loop(0, CH, step=CHG)
        def _(c):
            base = wid * CH + c
            pltpu.sync_copy(d1_hbm.at[pl.ds(base, CHG)], i1_v)
            pltpu.sync_copy(d2_hbm.at[pl.ds(base, CHG)], i2_v)
            cp1 = pltpu.async_copy(y_hbm.at[i1_v], r1_v, sem)
            cp2 = pltpu.async_copy(y_hbm.at[i2_v], r2_v, sem2)
            cp1.wait()
            pltpu.sync_copy(r1_v, o1_hbm.at[pl.ds(base, CHG)])
            cp2.wait()
            pltpu.sync_copy(r2_v, o2_hbm.at[pl.ds(base, CHG)])

    return k(y, d1, d2)


def _shared_body(x_ref, h_ref, su_ref, sd_ref, base_ref):
    h = h_ref[...]
    u = lax.dot_general(h, su_ref[...], (((1,), (1,)), ((), ())),
                        preferred_element_type=F32)
    act = jax.nn.silu(u[:, :SFF]) * u[:, SFF:]
    out = lax.dot_general(act, sd_ref[...], (((1,), (1,)), ((), ())),
                          preferred_element_type=F32)
    base_ref[...] = x_ref[...] + out


def _shared(x, h, su, sd, half):
    off = half * (T // TB // 2)
    return pl.pallas_call(
        _shared_body,
        grid=(T // TB // 2,),
        in_specs=[
            pl.BlockSpec((TB, H), lambda b: (b + off, 0)),
            pl.BlockSpec((TB, H), lambda b: (b + off, 0)),
            pl.BlockSpec((2 * SFF, H), lambda b: (0, 0)),
            pl.BlockSpec((H, SFF), lambda b: (0, 0)),
        ],
        out_specs=pl.BlockSpec((TB, H), lambda b: (b, 0)),
        out_shape=jax.ShapeDtypeStruct((T // 2, H), F32),
    )(x, h, su, sd)


def _moe_compute(hs_ref, up_v, dn_v, y_ref):
    hs = hs_ref[...]
    u = lax.dot_general(hs, up_v[...], (((1,), (1,)), ((), ())),
                        preferred_element_type=F32)
    act = jax.nn.silu(u[:, :FF]) * u[:, FF:]
    y_ref[...] = lax.dot_general(act, dn_v[...], (((1,), (1,)), ((), ())),
                                 preferred_element_type=F32)


def _moe_body(be_ref, hs_ref, up_hbm, dn_hbm, y_ref,
              up0, dn0, up1, dn1, s0u, s0d, s1u, s1d, run_ref):
    b = pl.program_id(0)
    e = be_ref[b]
    prev_e = be_ref[jnp.maximum(b - 1, 0)]
    first = b == 0
    changed = first | (e != prev_e)

    @pl.when(first)
    def _():
        run_ref[0] = 0
        pltpu.make_async_copy(up_hbm.at[e], up0, s0u).start()
        pltpu.make_async_copy(dn_hbm.at[e], dn0, s0d).start()

    @pl.when(jnp.logical_not(first) & (e != prev_e))
    def _():
        run_ref[0] = run_ref[0] + 1

    parity = lax.rem(run_ref[0], 2)

    @pl.when(changed & (parity == 0))
    def _():
        pltpu.make_async_copy(up_hbm.at[e], up0, s0u).wait()
        pltpu.make_async_copy(dn_hbm.at[e], dn0, s0d).wait()

    @pl.when(changed & (parity == 1))
    def _():
        pltpu.make_async_copy(up_hbm.at[e], up1, s1u).wait()
        pltpu.make_async_copy(dn_hbm.at[e], dn1, s1d).wait()

    nxt = be_ref[jnp.minimum(b + 1, NBLK - 1)]
    pre = (b < NBLK - 1) & (nxt != e)

    @pl.when(pre & (parity == 0))
    def _():
        pltpu.make_async_copy(up_hbm.at[nxt], up1, s1u).start()
        pltpu.make_async_copy(dn_hbm.at[nxt], dn1, s1d).start()

    @pl.when(pre & (parity == 1))
    def _():
        pltpu.make_async_copy(up_hbm.at[nxt], up0, s0u).start()
        pltpu.make_async_copy(dn_hbm.at[nxt], dn0, s0d).start()

    @pl.when(parity == 0)
    def _():
        _moe_compute(hs_ref, up0, dn0, y_ref)

    @pl.when(parity == 1)
    def _():
        _moe_compute(hs_ref, up1, dn1, y_ref)


def _moe(be_flat, hs, up_w, down_w):
    grid_spec = pltpu.PrefetchScalarGridSpec(
        num_scalar_prefetch=1,
        grid=(NBLK,),
        in_specs=[
            pl.BlockSpec((MB, H), lambda b, be: (b, 0)),
            pl.BlockSpec(memory_space=pl.ANY),
            pl.BlockSpec(memory_space=pl.ANY),
        ],
        out_specs=pl.BlockSpec((MB, H), lambda b, be: (b, 0)),
        scratch_shapes=[
            pltpu.VMEM((2 * FF, H), F32),
            pltpu.VMEM((H, FF), F32),
            pltpu.VMEM((2 * FF, H), F32),
            pltpu.VMEM((H, FF), F32),
            pltpu.SemaphoreType.DMA,
            pltpu.SemaphoreType.DMA,
            pltpu.SemaphoreType.DMA,
            pltpu.SemaphoreType.DMA,
            pltpu.SMEM((1,), jnp.int32),
        ],
    )
    return pl.pallas_call(
        _moe_body,
        grid_spec=grid_spec,
        out_shape=jax.ShapeDtypeStruct((PAD_T, H), F32),
    )(be_flat, hs, up_w, down_w)


def _final_body(ba_ref, bb_ref, meta_ref, y1_ref, y2_ref, out_ref):
    b = pl.program_id(0)
    w1 = meta_ref[:, 4:5]
    w2 = meta_ref[:, 5:6]
    comb = w1 * y1_ref[...] + w2 * y2_ref[...]

    @pl.when(b < (T // TB // 2))
    def _():
        out_ref[...] = ba_ref[...] + comb

    @pl.when(b >= (T // TB // 2))
    def _():
        out_ref[...] = bb_ref[...] + comb


def _final(base_a, base_b, meta, y1, y2):
    hg = T // TB // 2
    return pl.pallas_call(
        _final_body,
        grid=(T // TB,),
        in_specs=[
            pl.BlockSpec((TB, H), lambda b: (jnp.minimum(b, hg - 1), 0)),
            pl.BlockSpec((TB, H), lambda b: (jnp.maximum(b, hg) - hg, 0)),
            pl.BlockSpec((TB, E), lambda b: (b, 0)),
            pl.BlockSpec((TB, H), lambda b: (b, 0)),
            pl.BlockSpec((TB, H), lambda b: (b, 0)),
        ],
        out_specs=pl.BlockSpec((TB, H), lambda b: (b, 0)),
        out_shape=jax.ShapeDtypeStruct((T, H), F32),
    )(base_a, base_b, meta, y1, y2)


def kernel(x, ffn_norm_w, gate_w, up_w, down_w, shared_up_w, shared_down_w):
    nw = ffn_norm_w.reshape(1, H)
    h, meta, cnt = _prologue(x, nw, gate_w)
    dest, be = _route(meta, cnt)
    d1 = dest[:, 0]
    d2 = dest[:, 1]
    be_flat = be.reshape(NBLK)
    hs = _sc_scatter(h, d1, d2)
    base_a = _shared(x, h, shared_up_w, shared_down_w, 0)
    hs, base_a = lax.optimization_barrier((hs, base_a))
    y = _moe(be_flat, hs, up_w, down_w)
    y1, y2 = _sc_gather(y, d1, d2)
    base_b = _shared(x, h, shared_up_w, shared_down_w, 1)
    return _final(base_a, base_b, meta, y1, y2)


# speedup vs baseline: 1.0842x; 1.0587x over previous
import functools

import jax
import jax.numpy as jnp
from jax import lax
from jax.experimental import pallas as pl
from jax.experimental.pallas import tpu as pltpu
from jax.experimental.pallas import tpu_sc as plsc

H = 1024
E = 8
FF = 1024
SFF = 2048
T = 2048
TB = 512
MB = 256
NBLK = (T * 2) // MB + E
PAD_T = NBLK * MB
EPS = 1e-6
SCALING = 1.0

F32 = jnp.float32
BF16 = jnp.bfloat16

NC = 2
NS = 16
NW = NC * NS
CH = T // NW
CHG = CH // 2


def _prologue_body(x_ref, nw_ref, gw_ref, h_ref, meta_ref, cnt_ref, carry):
    b = pl.program_id(0)

    @pl.when(b == 0)
    def _():
        carry[...] = jnp.zeros_like(carry)

    x = x_ref[...]
    var = jnp.mean(x * x, axis=1, keepdims=True)
    h = x * lax.rsqrt(var + EPS) * nw_ref[...]
    h_ref[...] = h

    logits = lax.dot_general(h, gw_ref[...], (((1,), (1,)), ((), ())),
                             precision=lax.Precision.HIGHEST)
    lane = lax.broadcasted_iota(jnp.int32, (TB, E), 1)
    m1 = jnp.max(logits, axis=1, keepdims=True)
    i1 = jnp.min(jnp.where(logits == m1, lane, E), axis=1, keepdims=True)
    oh1 = lane == i1
    l2 = jnp.where(oh1, -jnp.inf, logits)
    m2 = jnp.max(l2, axis=1, keepdims=True)
    i2 = jnp.min(jnp.where(l2 == m2, lane, E), axis=1, keepdims=True)
    oh2 = lane == i2

    t = jnp.exp(m2 - m1)
    w1 = (1.0 / (1.0 + t)) * SCALING
    w2 = (t / (1.0 + t)) * SCALING

    ohc = oh1.astype(F32) + oh2.astype(F32)
    r = lax.broadcasted_iota(jnp.int32, (TB, TB), 0)
    c = lax.broadcasted_iota(jnp.int32, (TB, TB), 1)
    strict = (r > c).astype(F32)
    sex = lax.dot_general(strict, ohc, (((1,), (0,)), ((), ()))) + carry[...]
    prior1 = jnp.sum(jnp.where(oh1, sex, 0.0), axis=1, keepdims=True)
    prior2 = jnp.sum(jnp.where(oh2, sex, 0.0), axis=1, keepdims=True)
    carry[...] = carry[...] + jnp.sum(ohc, axis=0, keepdims=True)
    cnt_ref[...] = carry[...]

    meta = jnp.zeros((TB, E), F32)
    for j, v in enumerate((i1.astype(F32), i2.astype(F32),
                           prior1, prior2, w1, w2)):
        meta = jnp.where(lane == j, v, meta)
    meta_ref[...] = meta


def _prologue(x, nw, gw):
    return pl.pallas_call(
        _prologue_body,
        grid=(T // TB,),
        in_specs=[
            pl.BlockSpec((TB, H), lambda b: (b, 0)),
            pl.BlockSpec((1, H), lambda b: (0, 0)),
            pl.BlockSpec((E, H), lambda b: (0, 0)),
        ],
        out_specs=[
            pl.BlockSpec((TB, H), lambda b: (b, 0)),
            pl.BlockSpec((TB, E), lambda b: (b, 0)),
            pl.BlockSpec((1, E), lambda b: (0, 0)),
        ],
        out_shape=[
            jax.ShapeDtypeStruct((T, H), F32),
            jax.ShapeDtypeStruct((T, E), F32),
            jax.ShapeDtypeStruct((1, E), F32),
        ],
        scratch_shapes=[pltpu.VMEM((1, E), F32)],
    )(x, nw, gw)


def _route_body(meta_ref, cnt_ref, d1_ref, d2_ref, be_ref):
    lane8 = lax.broadcasted_iota(jnp.int32, (T, E), 1)
    bcol = lax.broadcasted_iota(jnp.int32, (1, NBLK + 1), 1)
    bstart = bcol * MB
    padded_row = jnp.zeros((T, E), F32)
    nblocks = jnp.zeros((1, NBLK + 1), jnp.int32)
    run = 0
    for g in range(E):
        cg = cnt_ref[0, g].astype(jnp.int32)
        pg = ((cg + (MB - 1)) // MB) * MB
        padded_row = jnp.where(lane8 == g, pg.astype(F32), padded_row)
        nblocks = nblocks + jnp.where(run <= bstart, 1, 0)
        run = run + pg
    be = jnp.maximum(nblocks - 1, 0)
    be_ref[...] = jnp.where(bcol == NBLK, run // MB, be)

    meta = meta_ref[...]
    lane = lane8.astype(F32)
    i1 = meta[:, 0:1]
    i2 = meta[:, 1:2]
    b1 = jnp.sum(jnp.where(lane < i1, padded_row, 0.0), axis=1, keepdims=True)
    b2 = jnp.sum(jnp.where(lane < i2, padded_row, 0.0), axis=1, keepdims=True)
    d1_ref[...] = (b1 + meta[:, 2:3]).astype(jnp.int32)
    d2_ref[...] = (b2 + meta[:, 3:4]).astype(jnp.int32)


def _route(meta, cnt):
    return pl.pallas_call(
        _route_body,
        grid=(1,),
        in_specs=[
            pl.BlockSpec((T, E), lambda i: (0, 0)),
            pl.BlockSpec(memory_space=pltpu.SMEM),
        ],
        out_specs=[
            pl.BlockSpec((T, 1), lambda i: (0, 0)),
            pl.BlockSpec((T, 1), lambda i: (0, 0)),
            pl.BlockSpec((1, NBLK + 1), lambda i: (0, 0)),
        ],
        out_shape=[
            jax.ShapeDtypeStruct((T, 1), jnp.int32),
            jax.ShapeDtypeStruct((T, 1), jnp.int32),
            jax.ShapeDtypeStruct((1, NBLK + 1), jnp.int32),
        ],
    )(meta, cnt)


def _sc_scatter(h, d1, d2):
    mesh = plsc.VectorSubcoreMesh(core_axis_name="c", subcore_axis_name="s")

    @functools.partial(
        pl.kernel,
        mesh=mesh,
        out_type=jax.ShapeDtypeStruct((PAD_T, H), F32),
        scratch_types=[
            pltpu.VMEM((CH,), jnp.int32),
            pltpu.VMEM((CH,), jnp.int32),
            pltpu.VMEM((CH, H), F32),
            pltpu.SemaphoreType.DMA,
            pltpu.SemaphoreType.DMA,
        ],
    )
    def k(h_hbm, d1_hbm, d2_hbm, hs_hbm, i1_v, i2_v, rows_v, sem, sem2):
        wid = lax.axis_index("s") * NC + lax.axis_index("c")
        base = wid * CH
        pltpu.sync_copy(d1_hbm.at[pl.ds(base, CH)], i1_v)
        pltpu.sync_copy(d2_hbm.at[pl.ds(base, CH)], i2_v)
        pltpu.sync_copy(h_hbm.at[pl.ds(base, CH)], rows_v)
        cp1 = pltpu.async_copy(rows_v, hs_hbm.at[i1_v], sem)
        cp2 = pltpu.async_copy(rows_v, hs_hbm.at[i2_v], sem2)
        cp1.wait()
        cp2.wait()

    return k(h, d1, d2)


def _sc_gather(y, d1, d2):
    mesh = plsc.VectorSubcoreMesh(core_axis_name="c", subcore_axis_name="s")

    @functools.partial(
        pl.kernel,
        mesh=mesh,
        out_type=[
            jax.ShapeDtypeStruct((T, H), F32),
            jax.ShapeDtypeStruct((T, H), F32),
        ],
        scratch_types=[
            pltpu.VMEM((CHG,), jnp.int32),
            pltpu.VMEM((CHG,), jnp.int32),
            pltpu.VMEM((CHG, H), F32),
            pltpu.VMEM((CHG, H), F32),
            pltpu.SemaphoreType.DMA,
            pltpu.SemaphoreType.DMA,
        ],
    )
    def k(y_hbm, d1_hbm, d2_hbm, o1_hbm, o2_hbm, i1_v, i2_v, r1_v, r2_v,
          sem, sem2):
        wid = lax.axis_index("s") * NC + lax.axis_index("c")

        @pl.loop(0, CH, step=CHG)
        def _(c):
            base = wid * CH + c
            pltpu.sync_copy(d1_hbm.at[pl.ds(base, CHG)], i1_v)
            pltpu.sync_copy(d2_hbm.at[pl.ds(base, CHG)], i2_v)
            cp1 = pltpu.async_copy(y_hbm.at[i1_v], r1_v, sem)
            cp2 = pltpu.async_copy(y_hbm.at[i2_v], r2_v, sem2)
            cp1.wait()
            pltpu.sync_copy(r1_v, o1_hbm.at[pl.ds(base, CHG)])
            cp2.wait()
            pltpu.sync_copy(r2_v, o2_hbm.at[pl.ds(base, CHG)])

    return k(y, d1, d2)


def _shared_body(x_ref, h_ref, su_ref, sd_ref, base_ref):
    h = h_ref[...]
    u = lax.dot_general(h, su_ref[...], (((1,), (1,)), ((), ())),
                        preferred_element_type=F32)
    act = jax.nn.silu(u[:, :SFF]) * u[:, SFF:]
    out = lax.dot_general(act, sd_ref[...], (((1,), (1,)), ((), ())),
                          preferred_element_type=F32)
    base_ref[...] = x_ref[...] + out


def _shared(x, h, su, sd):
    return pl.pallas_call(
        _shared_body,
        grid=(T // TB,),
        in_specs=[
            pl.BlockSpec((TB, H), lambda b: (b, 0)),
            pl.BlockSpec((TB, H), lambda b: (b, 0)),
            pl.BlockSpec((2 * SFF, H), lambda b: (0, 0)),
            pl.BlockSpec((H, SFF), lambda b: (0, 0)),
        ],
        out_specs=pl.BlockSpec((TB, H), lambda b: (b, 0)),
        out_shape=jax.ShapeDtypeStruct((T, H), F32),
    )(x, h, su, sd)


def _moe_compute(hs_ref, up_v, dn_v, y_ref):
    hs = hs_ref[...]
    u = lax.dot_general(hs, up_v[...], (((1,), (1,)), ((), ())),
                        preferred_element_type=F32)
    act = jax.nn.silu(u[:, :FF]) * u[:, FF:]
    y_ref[...] = lax.dot_general(act, dn_v[...], (((1,), (1,)), ((), ())),
                                 preferred_element_type=F32)


def _moe_body(be_ref, hs_ref, up_hbm, dn_hbm, y_ref,
              up0, dn0, up1, dn1, s0u, s0d, s1u, s1d, run_ref):
    b = pl.program_id(0)
    e = be_ref[b]
    prev_e = be_ref[jnp.maximum(b - 1, 0)]
    first = b == 0
    changed = first | (e != prev_e)

    @pl.when(first)
    def _():
        run_ref[0] = 0
        pltpu.make_async_copy(up_hbm.at[e], up0, s0u).start()
        pltpu.make_async_copy(dn_hbm.at[e], dn0, s0d).start()

    @pl.when(jnp.logical_not(first) & (e != prev_e))
    def _():
        run_ref[0] = run_ref[0] + 1

    parity = lax.rem(run_ref[0], 2)

    @pl.when(changed & (parity == 0))
    def _():
        pltpu.make_async_copy(up_hbm.at[e], up0, s0u).wait()
        pltpu.make_async_copy(dn_hbm.at[e], dn0, s0d).wait()

    @pl.when(changed & (parity == 1))
    def _():
        pltpu.make_async_copy(up_hbm.at[e], up1, s1u).wait()
        pltpu.make_async_copy(dn_hbm.at[e], dn1, s1d).wait()

    nxt = be_ref[jnp.minimum(b + 1, NBLK - 1)]
    pre = (b < NBLK - 1) & (nxt != e)

    @pl.when(pre & (parity == 0))
    def _():
        pltpu.make_async_copy(up_hbm.at[nxt], up1, s1u).start()
        pltpu.make_async_copy(dn_hbm.at[nxt], dn1, s1d).start()

    @pl.when(pre & (parity == 1))
    def _():
        pltpu.make_async_copy(up_hbm.at[nxt], up0, s0u).start()
        pltpu.make_async_copy(dn_hbm.at[nxt], dn0, s0d).start()

    real = b < be_ref[NBLK]

    @pl.when(real & (parity == 0))
    def _():
        _moe_compute(hs_ref, up0, dn0, y_ref)

    @pl.when(real & (parity == 1))
    def _():
        _moe_compute(hs_ref, up1, dn1, y_ref)


def _moe(be_flat, hs, up_w, down_w):
    grid_spec = pltpu.PrefetchScalarGridSpec(
        num_scalar_prefetch=1,
        grid=(NBLK,),
        in_specs=[
            pl.BlockSpec((MB, H), lambda b, be: (b, 0)),
            pl.BlockSpec(memory_space=pl.ANY),
            pl.BlockSpec(memory_space=pl.ANY),
        ],
        out_specs=pl.BlockSpec((MB, H), lambda b, be: (b, 0)),
        scratch_shapes=[
            pltpu.VMEM((2 * FF, H), F32),
            pltpu.VMEM((H, FF), F32),
            pltpu.VMEM((2 * FF, H), F32),
            pltpu.VMEM((H, FF), F32),
            pltpu.SemaphoreType.DMA,
            pltpu.SemaphoreType.DMA,
            pltpu.SemaphoreType.DMA,
            pltpu.SemaphoreType.DMA,
            pltpu.SMEM((1,), jnp.int32),
        ],
    )
    return pl.pallas_call(
        _moe_body,
        grid_spec=grid_spec,
        out_shape=jax.ShapeDtypeStruct((PAD_T, H), F32),
    )(be_flat, hs, up_w, down_w)


def _final_body(base_ref, meta_ref, y1_ref, y2_ref, out_ref):
    w1 = meta_ref[:, 4:5]
    w2 = meta_ref[:, 5:6]
    out_ref[...] = base_ref[...] + w1 * y1_ref[...] + w2 * y2_ref[...]


def _final(base, meta, y1, y2):
    return pl.pallas_call(
        _final_body,
        grid=(T // TB,),
        in_specs=[
            pl.BlockSpec((TB, H), lambda b: (b, 0)),
            pl.BlockSpec((TB, E), lambda b: (b, 0)),
            pl.BlockSpec((TB, H), lambda b: (b, 0)),
            pl.BlockSpec((TB, H), lambda b: (b, 0)),
        ],
        out_specs=pl.BlockSpec((TB, H), lambda b: (b, 0)),
        out_shape=jax.ShapeDtypeStruct((T, H), F32),
    )(base, meta, y1, y2)


def kernel(x, ffn_norm_w, gate_w, up_w, down_w, shared_up_w, shared_down_w):
    nw = ffn_norm_w.reshape(1, H)
    h, meta, cnt = _prologue(x, nw, gate_w)
    d1a, d2a, be = _route(meta, cnt)
    d1 = d1a.reshape(T)
    d2 = d2a.reshape(T)
    be_flat = be.reshape(NBLK + 1)
    hs = _sc_scatter(h, d1, d2)
    y = _moe(be_flat, hs, up_w, down_w)
    y1, y2 = _sc_gather(y, d1, d2)
    base = _shared(x, h, shared_up_w, shared_down_w)
    return _final(base, meta, y1, y2)


# speedup vs baseline: 1.1102x; 1.0239x over previous
import functools

import jax
import jax.numpy as jnp
from jax import lax
from jax.experimental import pallas as pl
from jax.experimental.pallas import tpu as pltpu
from jax.experimental.pallas import tpu_sc as plsc

H = 1024
E = 8
FF = 1024
SFF = 2048
T = 2048
TB = 512
MB = 256
NBLK = (T * 2) // MB + E
PAD_T = NBLK * MB
EPS = 1e-6
SCALING = 1.0

F32 = jnp.float32
BF16 = jnp.bfloat16

NC = 2
NS = 16
NW = NC * NS
CH = T // NW
CHG = CH // 2


def _prologue_body(x_ref, nw_ref, gw_ref, h_ref, meta_ref, cnt_ref, carry):
    b = pl.program_id(0)

    @pl.when(b == 0)
    def _():
        carry[...] = jnp.zeros_like(carry)

    x = x_ref[...]
    var = jnp.mean(x * x, axis=1, keepdims=True)
    h = x * lax.rsqrt(var + EPS) * nw_ref[...]
    h_ref[...] = h

    logits = lax.dot_general(h, gw_ref[...], (((1,), (1,)), ((), ())))
    lane = lax.broadcasted_iota(jnp.int32, (TB, E), 1)
    m1 = jnp.max(logits, axis=1, keepdims=True)
    i1 = jnp.min(jnp.where(logits == m1, lane, E), axis=1, keepdims=True)
    oh1 = lane == i1
    l2 = jnp.where(oh1, -jnp.inf, logits)
    m2 = jnp.max(l2, axis=1, keepdims=True)
    i2 = jnp.min(jnp.where(l2 == m2, lane, E), axis=1, keepdims=True)
    oh2 = lane == i2

    t = jnp.exp(m2 - m1)
    w1 = (1.0 / (1.0 + t)) * SCALING
    w2 = (t / (1.0 + t)) * SCALING

    ohc = oh1.astype(F32) + oh2.astype(F32)
    r = lax.broadcasted_iota(jnp.int32, (TB, TB), 0)
    c = lax.broadcasted_iota(jnp.int32, (TB, TB), 1)
    strict = (r > c).astype(F32)
    sex = lax.dot_general(strict, ohc, (((1,), (0,)), ((), ()))) + carry[...]
    prior1 = jnp.sum(jnp.where(oh1, sex, 0.0), axis=1, keepdims=True)
    prior2 = jnp.sum(jnp.where(oh2, sex, 0.0), axis=1, keepdims=True)
    carry[...] = carry[...] + jnp.sum(ohc, axis=0, keepdims=True)
    cnt_ref[...] = carry[...]

    meta = jnp.zeros((TB, E), F32)
    for j, v in enumerate((i1.astype(F32), i2.astype(F32),
                           prior1, prior2, w1, w2)):
        meta = jnp.where(lane == j, v, meta)
    meta_ref[...] = meta


def _prologue(x, nw, gw):
    return pl.pallas_call(
        _prologue_body,
        grid=(T // TB,),
        in_specs=[
            pl.BlockSpec((TB, H), lambda b: (b, 0)),
            pl.BlockSpec((1, H), lambda b: (0, 0)),
            pl.BlockSpec((E, H), lambda b: (0, 0)),
        ],
        out_specs=[
            pl.BlockSpec((TB, H), lambda b: (b, 0)),
            pl.BlockSpec((TB, E), lambda b: (b, 0)),
            pl.BlockSpec((1, E), lambda b: (0, 0)),
        ],
        out_shape=[
            jax.ShapeDtypeStruct((T, H), F32),
            jax.ShapeDtypeStruct((T, E), F32),
            jax.ShapeDtypeStruct((1, E), F32),
        ],
        scratch_shapes=[pltpu.VMEM((1, E), F32)],
    )(x, nw, gw)


def _route_body(meta_ref, cnt_ref, d1_ref, d2_ref, be_ref):
    lane8 = lax.broadcasted_iota(jnp.int32, (T, E), 1)
    bcol = lax.broadcasted_iota(jnp.int32, (1, NBLK + 1), 1)
    bstart = bcol * MB
    padded_row = jnp.zeros((T, E), F32)
    nblocks = jnp.zeros((1, NBLK + 1), jnp.int32)
    run = 0
    for g in range(E):
        cg = cnt_ref[0, g].astype(jnp.int32)
        pg = ((cg + (MB - 1)) // MB) * MB
        padded_row = jnp.where(lane8 == g, pg.astype(F32), padded_row)
        nblocks = nblocks + jnp.where(run <= bstart, 1, 0)
        run = run + pg
    be = jnp.maximum(nblocks - 1, 0)
    be_ref[...] = jnp.where(bcol == NBLK, run // MB, be)

    meta = meta_ref[...]
    lane = lane8.astype(F32)
    i1 = meta[:, 0:1]
    i2 = meta[:, 1:2]
    b1 = jnp.sum(jnp.where(lane < i1, padded_row, 0.0), axis=1, keepdims=True)
    b2 = jnp.sum(jnp.where(lane < i2, padded_row, 0.0), axis=1, keepdims=True)
    d1_ref[...] = (b1 + meta[:, 2:3]).astype(jnp.int32)
    d2_ref[...] = (b2 + meta[:, 3:4]).astype(jnp.int32)


def _route(meta, cnt):
    return pl.pallas_call(
        _route_body,
        grid=(1,),
        in_specs=[
            pl.BlockSpec((T, E), lambda i: (0, 0)),
            pl.BlockSpec(memory_space=pltpu.SMEM),
        ],
        out_specs=[
            pl.BlockSpec((T, 1), lambda i: (0, 0)),
            pl.BlockSpec((T, 1), lambda i: (0, 0)),
            pl.BlockSpec((1, NBLK + 1), lambda i: (0, 0)),
        ],
        out_shape=[
            jax.ShapeDtypeStruct((T, 1), jnp.int32),
            jax.ShapeDtypeStruct((T, 1), jnp.int32),
            jax.ShapeDtypeStruct((1, NBLK + 1), jnp.int32),
        ],
    )(meta, cnt)


def _sc_scatter(h, d1, d2):
    mesh = plsc.VectorSubcoreMesh(core_axis_name="c", subcore_axis_name="s")

    @functools.partial(
        pl.kernel,
        mesh=mesh,
        out_type=jax.ShapeDtypeStruct((PAD_T, H), F32),
        scratch_types=[
            pltpu.VMEM((CH,), jnp.int32),
            pltpu.VMEM((CH,), jnp.int32),
            pltpu.VMEM((CH, H), F32),
            pltpu.SemaphoreType.DMA,
            pltpu.SemaphoreType.DMA,
        ],
    )
    def k(h_hbm, d1_hbm, d2_hbm, hs_hbm, i1_v, i2_v, rows_v, sem, sem2):
        wid = lax.axis_index("s") * NC + lax.axis_index("c")
        base = wid * CH
        pltpu.sync_copy(d1_hbm.at[pl.ds(base, CH)], i1_v)
        pltpu.sync_copy(d2_hbm.at[pl.ds(base, CH)], i2_v)
        pltpu.sync_copy(h_hbm.at[pl.ds(base, CH)], rows_v)
        cp1 = pltpu.async_copy(rows_v, hs_hbm.at[i1_v], sem)
        cp2 = pltpu.async_copy(rows_v, hs_hbm.at[i2_v], sem2)
        cp1.wait()
        cp2.wait()

    return k(h, d1, d2)


def _sc_gather(y, d1, d2):
    mesh = plsc.VectorSubcoreMesh(core_axis_name="c", subcore_axis_name="s")

    @functools.partial(
        pl.kernel,
        mesh=mesh,
        out_type=[
            jax.ShapeDtypeStruct((T, H), F32),
            jax.ShapeDtypeStruct((T, H), F32),
        ],
        scratch_types=[
            pltpu.VMEM((CHG,), jnp.int32),
            pltpu.VMEM((CHG,), jnp.int32),
            pltpu.VMEM((CHG, H), F32),
            pltpu.VMEM((CHG, H), F32),
            pltpu.SemaphoreType.DMA,
            pltpu.SemaphoreType.DMA,
        ],
    )
    def k(y_hbm, d1_hbm, d2_hbm, o1_hbm, o2_hbm, i1_v, i2_v, r1_v, r2_v,
          sem, sem2):
        wid = lax.axis_index("s") * NC + lax.axis_index("c")

        @pl.loop(0, CH, step=CHG)
        def _(c):
            base = wid * CH + c
            pltpu.sync_copy(d1_hbm.at[pl.ds(base, CHG)], i1_v)
            pltpu.sync_copy(d2_hbm.at[pl.ds(base, CHG)], i2_v)
            cp1 = pltpu.async_copy(y_hbm.at[i1_v], r1_v, sem)
            cp2 = pltpu.async_copy(y_hbm.at[i2_v], r2_v, sem2)
            cp1.wait()
            pltpu.sync_copy(r1_v, o1_hbm.at[pl.ds(base, CHG)])
            cp2.wait()
            pltpu.sync_copy(r2_v, o2_hbm.at[pl.ds(base, CHG)])

    return k(y, d1, d2)


def _shared_body(x_ref, h_ref, su_ref, sd_ref, base_ref):
    h = h_ref[...]
    u = lax.dot_general(h, su_ref[...], (((1,), (1,)), ((), ())),
                        preferred_element_type=F32)
    act = jax.nn.silu(u[:, :SFF]) * u[:, SFF:]
    out = lax.dot_general(act, sd_ref[...], (((1,), (1,)), ((), ())),
                          preferred_element_type=F32)
    base_ref[...] = x_ref[...] + out


def _shared(x, h, su, sd):
    return pl.pallas_call(
        _shared_body,
        grid=(T // TB,),
        in_specs=[
            pl.BlockSpec((TB, H), lambda b: (b, 0)),
            pl.BlockSpec((TB, H), lambda b: (b, 0)),
            pl.BlockSpec((2 * SFF, H), lambda b: (0, 0)),
            pl.BlockSpec((H, SFF), lambda b: (0, 0)),
        ],
        out_specs=pl.BlockSpec((TB, H), lambda b: (b, 0)),
        out_shape=jax.ShapeDtypeStruct((T, H), F32),
    )(x, h, su, sd)


def _moe_compute(hs_ref, up_v, dn_v, y_ref):
    hs = hs_ref[...]
    u = lax.dot_general(hs, up_v[...], (((1,), (1,)), ((), ())),
                        preferred_element_type=F32)
    act = jax.nn.silu(u[:, :FF]) * u[:, FF:]
    y_ref[...] = lax.dot_general(act, dn_v[...], (((1,), (1,)), ((), ())),
                                 preferred_element_type=F32)


def _moe_body(be_ref, hs_ref, up_hbm, dn_hbm, y_ref,
              up0, dn0, up1, dn1, s0u, s0d, s1u, s1d, run_ref):
    b = pl.program_id(0)
    e = be_ref[b]
    prev_e = be_ref[jnp.maximum(b - 1, 0)]
    first = b == 0
    changed = first | (e != prev_e)

    @pl.when(first)
    def _():
        run_ref[0] = 0
        pltpu.make_async_copy(up_hbm.at[e], up0, s0u).start()
        pltpu.make_async_copy(dn_hbm.at[e], dn0, s0d).start()

    @pl.when(jnp.logical_not(first) & (e != prev_e))
    def _():
        run_ref[0] = run_ref[0] + 1

    parity = lax.rem(run_ref[0], 2)

    @pl.when(changed & (parity == 0))
    def _():
        pltpu.make_async_copy(up_hbm.at[e], up0, s0u).wait()
        pltpu.make_async_copy(dn_hbm.at[e], dn0, s0d).wait()

    @pl.when(changed & (parity == 1))
    def _():
        pltpu.make_async_copy(up_hbm.at[e], up1, s1u).wait()
        pltpu.make_async_copy(dn_hbm.at[e], dn1, s1d).wait()

    nxt = be_ref[jnp.minimum(b + 1, NBLK - 1)]
    pre = (b < NBLK - 1) & (nxt != e)

    @pl.when(pre & (parity == 0))
    def _():
        pltpu.make_async_copy(up_hbm.at[nxt], up1, s1u).start()
        pltpu.make_async_copy(dn_hbm.at[nxt], dn1, s1d).start()

    @pl.when(pre & (parity == 1))
    def _():
        pltpu.make_async_copy(up_hbm.at[nxt], up0, s0u).start()
        pltpu.make_async_copy(dn_hbm.at[nxt], dn0, s0d).start()

    real = b < be_ref[NBLK]

    @pl.when(real & (parity == 0))
    def _():
        _moe_compute(hs_ref, up0, dn0, y_ref)

    @pl.when(real & (parity == 1))
    def _():
        _moe_compute(hs_ref, up1, dn1, y_ref)


def _moe(be_flat, hs, up_w, down_w):
    grid_spec = pltpu.PrefetchScalarGridSpec(
        num_scalar_prefetch=1,
        grid=(NBLK,),
        in_specs=[
            pl.BlockSpec((MB, H), lambda b, be: (b, 0)),
            pl.BlockSpec(memory_space=pl.ANY),
            pl.BlockSpec(memory_space=pl.ANY),
        ],
        out_specs=pl.BlockSpec((MB, H), lambda b, be: (b, 0)),
        scratch_shapes=[
            pltpu.VMEM((2 * FF, H), F32),
            pltpu.VMEM((H, FF), F32),
            pltpu.VMEM((2 * FF, H), F32),
            pltpu.VMEM((H, FF), F32),
            pltpu.SemaphoreType.DMA,
            pltpu.SemaphoreType.DMA,
            pltpu.SemaphoreType.DMA,
            pltpu.SemaphoreType.DMA,
            pltpu.SMEM((1,), jnp.int32),
        ],
    )
    return pl.pallas_call(
        _moe_body,
        grid_spec=grid_spec,
        out_shape=jax.ShapeDtypeStruct((PAD_T, H), F32),
    )(be_flat, hs, up_w, down_w)


def _final_body(base_ref, meta_ref, y1_ref, y2_ref, out_ref):
    w1 = meta_ref[:, 4:5]
    w2 = meta_ref[:, 5:6]
    out_ref[...] = base_ref[...] + w1 * y1_ref[...] + w2 * y2_ref[...]


def _final(base, meta, y1, y2):
    return pl.pallas_call(
        _final_body,
        grid=(T // TB,),
        in_specs=[
            pl.BlockSpec((TB, H), lambda b: (b, 0)),
            pl.BlockSpec((TB, E), lambda b: (b, 0)),
            pl.BlockSpec((TB, H), lambda b: (b, 0)),
            pl.BlockSpec((TB, H), lambda b: (b, 0)),
        ],
        out_specs=pl.BlockSpec((TB, H), lambda b: (b, 0)),
        out_shape=jax.ShapeDtypeStruct((T, H), F32),
    )(base, meta, y1, y2)


def kernel(x, ffn_norm_w, gate_w, up_w, down_w, shared_up_w, shared_down_w):
    nw = ffn_norm_w.reshape(1, H)
    h, meta, cnt = _prologue(x, nw, gate_w)
    d1a, d2a, be = _route(meta, cnt)
    d1 = d1a.reshape(T)
    d2 = d2a.reshape(T)
    be_flat = be.reshape(NBLK + 1)
    hs = _sc_scatter(h, d1, d2)
    y = _moe(be_flat, hs, up_w, down_w)
    y1, y2 = _sc_gather(y, d1, d2)
    base = _shared(x, h, shared_up_w, shared_down_w)
    return _final(base, meta, y1, y2)
